# baseline (speedup 1.0000x reference)
import jax
import jax.numpy as jnp
from jax import lax
from jax.experimental import pallas as pl
from jax.experimental.pallas import tpu as pltpu, tpu_sc as plsc

_N_VAL = 1000
_ROWS = 4096 * 20
_NW = 32
_RPW = _ROWS // _NW
_IC = _RPW // 128

_FR, _FC = 640, 128000
_BR, _BC = 128, 16000


def _fill_body(o_ref):
    o_ref[...] = jnp.zeros((_BR, _BC), jnp.float32)


def _sc_body(x_hbm, out_ref, idx_v, ones_v, ssem):
    wid = lax.axis_index("s") * 2 + lax.axis_index("c")
    base_row = wid * _RPW

    pltpu.sync_copy(x_hbm.at[wid], idx_v)

    lane = lax.iota(jnp.int32, 16)
    ones16 = jnp.full((16,), 1.0, jnp.float32)
    for k in range(8):
        ones_v[pl.ds(k * 16, 16)] = ones16

    def pos_body(j, carry):
        r = j * 16
        xv = idx_v[r // 128, pl.ds(r % 128, 16)]
        gpos = (base_row + r + lane) * _N_VAL + xv
        idx_v[r // 128, pl.ds(r % 128, 16)] = gpos
        return carry
    lax.fori_loop(0, _RPW // 16, pos_body, 0)

    def scat(j, carry):
        pltpu.async_copy(ones_v, out_ref.at[idx_v.at[j]], ssem)
        return carry
    lax.fori_loop(0, _IC, scat, 0)

    def sdrain(j, carry):
        pltpu.make_async_copy(ones_v, out_ref.at[idx_v.at[j]], ssem).wait()
        return carry
    lax.fori_loop(0, _IC, sdrain, 0)


def kernel(x):
    zeros2d = pl.pallas_call(
        _fill_body,
        grid=(_FR // _BR, _FC // _BC),
        out_specs=pl.BlockSpec((_BR, _BC), lambda i, j: (i, j)),
        out_shape=jax.ShapeDtypeStruct((_FR, _FC), jnp.float32),
    )()
    x3 = x.reshape(_NW, _RPW // 128, 128)
    buf = jax.new_ref(zeros2d.reshape(_ROWS * _N_VAL))
    mesh = plsc.VectorSubcoreMesh(core_axis_name="c", subcore_axis_name="s")
    scatter_ones = pl.kernel(
        _sc_body,
        mesh=mesh,
        scratch_types=[
            pltpu.VMEM((_RPW // 128, 128), jnp.int32),
            pltpu.VMEM((128,), jnp.float32),
            pltpu.SemaphoreType.DMA,
        ],
        compiler_params=pltpu.CompilerParams(needs_layout_passes=False),
    )
    scatter_ones(x3, buf)
    return buf[...].reshape(x.shape + (_N_VAL,))


# speedup vs baseline: 1.0052x; 1.0052x over previous
import jax
import jax.numpy as jnp
from jax import lax
from jax.experimental import pallas as pl
from jax.experimental.pallas import tpu as pltpu, tpu_sc as plsc
from jax._src.pallas import mpmd as _pl_mpmd

_N_VAL = 1000
_ROWS = 4096 * 20
_NW = 32
_RPW = _ROWS // _NW
_IC = _RPW // 128

_FR, _FC = 640, 128000
_BR, _BC = 128, 16000


def _fill_body(o_ref):
    o_ref[...] = jnp.zeros((_BR, _BC), jnp.float32)


def _sc_body(x_hbm, zeroed_in, out_ref, idx_v, ones_v, ssem):
    del zeroed_in
    wid = lax.axis_index("s") * 2 + lax.axis_index("c")
    base_row = wid * _RPW

    pltpu.sync_copy(x_hbm.at[wid], idx_v)

    lane = lax.iota(jnp.int32, 16)
    ones16 = jnp.full((16,), 1.0, jnp.float32)
    for k in range(8):
        ones_v[pl.ds(k * 16, 16)] = ones16

    def pos_body(j, carry):
        r = j * 16
        xv = idx_v[r // 128, pl.ds(r % 128, 16)]
        gpos = (base_row + r + lane) * _N_VAL + xv
        idx_v[r // 128, pl.ds(r % 128, 16)] = gpos
        return carry
    lax.fori_loop(0, _RPW // 16, pos_body, 0)

    def scat(j, carry):
        pltpu.async_copy(ones_v, out_ref.at[idx_v.at[j]], ssem)
        return carry
    lax.fori_loop(0, _IC, scat, 0)

    def sdrain(j, carry):
        pltpu.make_async_copy(ones_v, out_ref.at[idx_v.at[j]], ssem).wait()
        return carry
    lax.fori_loop(0, _IC, sdrain, 0)


def kernel(x):
    zeros2d = pl.pallas_call(
        _fill_body,
        grid=(_FR // _BR, _FC // _BC),
        out_specs=pl.BlockSpec((_BR, _BC), lambda i, j: (i, j)),
        out_shape=jax.ShapeDtypeStruct((_FR, _FC), jnp.float32),
    )()
    x3 = x.reshape(_NW, _RPW // 128, 128)
    mesh = plsc.VectorSubcoreMesh(core_axis_name="c", subcore_axis_name="s")
    scatter_ones = _pl_mpmd._mpmd_map(
        [(mesh, _sc_body)],
        jax.ShapeDtypeStruct((_ROWS * _N_VAL,), jnp.float32),
        input_output_aliases={1: 0},
        scratch_types=[
            pltpu.VMEM((_RPW // 128, 128), jnp.int32),
            pltpu.VMEM((128,), jnp.float32),
            pltpu.SemaphoreType.DMA,
        ],
        compiler_params=pltpu.CompilerParams(needs_layout_passes=False),
    )
    out = scatter_ones(x3, zeros2d.reshape(_ROWS * _N_VAL))
    return out.reshape(x.shape + (_N_VAL,))


# speedup vs baseline: 1.6490x; 1.6405x over previous
import jax
import jax.numpy as jnp
from jax import lax
from jax.experimental import pallas as pl
from jax.experimental.pallas import tpu as pltpu, tpu_sc as plsc

_N_VAL = 1000
_NR, _NC = 4096, 20
_NKB = 8
_PC = 256
_NW = 32
_IPW = _NR // _NW
_VPW = _IPW * _NC

_EB = 64


def _sc_body(x_hbm, zeros_hbm, pos_hbm, idx_v, slab_v):
    wid = lax.axis_index("s") * 2 + lax.axis_index("c")
    base_i = wid * _IPW

    pltpu.sync_copy(x_hbm.at[pl.ds(base_i * _NC, _VPW)], idx_v)
    pltpu.sync_copy(zeros_hbm, slab_v)

    lane = lax.iota(jnp.int32, 16)

    def scat(v, carry):
        flat = v * 16 + lane
        xv = idx_v[pl.ds(v * 16, 16)]
        row = flat // _NC
        col = (flat - row * _NC) * _NKB + lax.shift_right_logical(xv, 7)
        val = lax.bitwise_and(xv, 127) + 1
        plsc.store_scatter(slab_v, [row, col], val)
        return carry
    lax.fori_loop(0, _VPW // 16, scat, 0)

    pltpu.sync_copy(slab_v, pos_hbm.at[pl.ds(base_i, _IPW)])


def _expand_body(pos_ref, o_ref):
    iota1 = lax.broadcasted_iota(jnp.int32, (_EB, 128), 1) + 1
    for j in range(_NC):
        for kb in range(_NKB):
            w = pos_ref[:, j * _NKB + kb]
            wb = jnp.broadcast_to(w[:, None], (_EB, 128))
            v = jnp.where(wb == iota1, 1.0, 0.0).astype(jnp.float32)
            kw = min(128, _N_VAL - kb * 128)
            o_ref[:, j, pl.ds(kb * 128, kw)] = v[:, :kw]


def kernel(x):
    xf = x.reshape(-1)
    zeros = jnp.zeros((_IPW, _PC), jnp.int32)
    mesh = plsc.VectorSubcoreMesh(core_axis_name="c", subcore_axis_name="s")
    posidx = pl.kernel(
        _sc_body,
        out_type=jax.ShapeDtypeStruct((_NR, _PC), jnp.int32),
        mesh=mesh,
        scratch_types=[
            pltpu.VMEM((_VPW,), jnp.int32),
            pltpu.VMEM((_IPW, _PC), jnp.int32),
        ],
        compiler_params=pltpu.CompilerParams(
            needs_layout_passes=False, use_tc_tiling_on_sc=True
        ),
    )(xf, zeros)
    return pl.pallas_call(
        _expand_body,
        grid=(_NR // _EB,),
        in_specs=[pl.BlockSpec((_EB, _PC), lambda i: (i, 0))],
        out_specs=pl.BlockSpec((_EB, _NC, _N_VAL), lambda i: (i, 0, 0)),
        out_shape=jax.ShapeDtypeStruct((_NR, _NC, _N_VAL), jnp.float32),
    )(posidx)


# speedup vs baseline: 2.0348x; 1.2339x over previous
import jax
import jax.numpy as jnp
from jax import lax
from jax.experimental import pallas as pl
from jax.experimental.pallas import tpu as pltpu, tpu_sc as plsc

_N_VAL = 1000
_NR, _NC = 4096, 20
_NKB = 8
_PC = 256
_NW = 32
_IPW = _NR // _NW
_VPW = _IPW * _NC

_EB = 64


def _sc_body(x_hbm, zeros_hbm, pos_hbm, idx_v, slab_v):
    wid = lax.axis_index("s") * 2 + lax.axis_index("c")
    base_i = wid * _IPW

    pltpu.sync_copy(x_hbm.at[pl.ds(base_i * _NC, _VPW)], idx_v)
    pltpu.sync_copy(zeros_hbm, slab_v)

    lane = lax.iota(jnp.int32, 16)

    def scat(v, carry):
        flat = v * 16 + lane
        xv = idx_v[pl.ds(v * 16, 16)]
        row = flat // _NC
        col = lax.shift_right_logical(xv, 7) * _NC + (flat - row * _NC)
        val = lax.bitwise_and(xv, 127) + 1
        plsc.store_scatter(slab_v, [row, col], val)
        return carry
    lax.fori_loop(0, _VPW // 16, scat, 0)

    pltpu.sync_copy(slab_v, pos_hbm.at[pl.ds(base_i, _IPW)])


def _expand_body(pos_ref, o_ref):
    for kb in range(_NKB):
        kw = min(128, _N_VAL - kb * 128)
        iota1 = lax.broadcasted_iota(jnp.int32, (_EB, _NC, kw), 2) + 1
        wb = pos_ref[:, pl.ds(kb * _NC, _NC)]
        v = jnp.where(wb[:, :, None] == iota1, 1.0, 0.0)
        o_ref[:, :, pl.ds(kb * 128, kw)] = v.astype(jnp.float32)


def kernel(x):
    xf = x.reshape(-1)
    zeros = jnp.zeros((_IPW, _PC), jnp.int32)
    mesh = plsc.VectorSubcoreMesh(core_axis_name="c", subcore_axis_name="s")
    posidx = pl.kernel(
        _sc_body,
        out_type=jax.ShapeDtypeStruct((_NR, _PC), jnp.int32),
        mesh=mesh,
        scratch_types=[
            pltpu.VMEM((_VPW,), jnp.int32),
            pltpu.VMEM((_IPW, _PC), jnp.int32),
        ],
        compiler_params=pltpu.CompilerParams(
            needs_layout_passes=False, use_tc_tiling_on_sc=True
        ),
    )(xf, zeros)
    return pl.pallas_call(
        _expand_body,
        grid=(_NR // _EB,),
        in_specs=[pl.BlockSpec((_EB, _PC), lambda i: (i, 0))],
        out_specs=pl.BlockSpec((_EB, _NC, _N_VAL), lambda i: (i, 0, 0)),
        out_shape=jax.ShapeDtypeStruct((_NR, _NC, _N_VAL), jnp.float32),
    )(posidx)
